# baseline (speedup 1.0000x reference)
import functools

import jax
import jax.numpy as jnp
from jax import lax
from jax.experimental import pallas as pl
from jax.experimental.pallas import tpu as pltpu
from jax.experimental.pallas import tpu_sc as plsc

N_NODES = 10000
N_EDGES = 320000
D_FEAT = 128
NUM_CLASSES = 2

_NC, _NS, _L = 2, 16, 16
_NW = _NC * _NS
_NT = N_EDGES // 128
_TPW = 79
_H0 = 40
_H1 = _TPW - _H0
_NP = 10240


def _proj_body(w_ref, x_ref, b_ref, q_ref):
    dims = (((1,), (1,)), ((), ()))
    d1 = lax.dot_general(w_ref[:, :D_FEAT], x_ref[...], dims,
                         preferred_element_type=jnp.float32)
    d2 = lax.dot_general(w_ref[:, D_FEAT:], x_ref[...], dims,
                         preferred_element_type=jnp.float32)

    def _pack16(row):
        return lax.bitcast_convert_type(
            row.astype(jnp.bfloat16), jnp.uint16).astype(jnp.int32)

    s0 = _pack16(d1[0:1] + b_ref[0])
    s1 = _pack16(d1[1:2] + b_ref[1])
    t0 = _pack16(d2[0:1])
    t1 = _pack16(d2[1:2])
    q_ref[0:1, :] = s0 | (s1 << 16)
    q_ref[1:2, :] = t0 | (t1 << 16)


_mesh = plsc.VectorSubcoreMesh(core_axis_name="c", subcore_axis_name="s")


@functools.partial(
    pl.kernel,
    mesh=_mesh,
    compiler_params=pltpu.CompilerParams(needs_layout_passes=False),
    out_type=jax.ShapeDtypeStruct((2 * N_EDGES,), jnp.float32),
    scratch_types=[
        pltpu.VMEM((2 * _NP,), jnp.int32),
        pltpu.VMEM((_TPW * 128,), jnp.int32),
        pltpu.VMEM((_TPW * 128,), jnp.int32),
        pltpu.VMEM((_TPW * 256,), jnp.float32),
        pltpu.SemaphoreType.DMA,
        pltpu.SemaphoreType.DMA,
        pltpu.SemaphoreType.DMA,
    ],
)
def _edge_score(q_hbm, ei_hbm, out_hbm, q_v, src_v, dst_v, out_v,
                sem_a, sem_b, sem_o):
    wid = lax.axis_index("s") * _NC + lax.axis_index("c")
    tlo = wid * (_NT - _TPW) // (_NW - 1)
    eb = tlo * 128
    cp = [
        pltpu.async_copy(q_hbm.at[0, :], q_v.at[pl.ds(0, _NP)], sem_a),
        pltpu.async_copy(q_hbm.at[1, :], q_v.at[pl.ds(_NP, _NP)], sem_a),
        pltpu.async_copy(ei_hbm.at[0, pl.ds(eb, _H0 * 128)],
                         src_v.at[pl.ds(0, _H0 * 128)], sem_a),
        pltpu.async_copy(ei_hbm.at[1, pl.ds(eb, _H0 * 128)],
                         dst_v.at[pl.ds(0, _H0 * 128)], sem_a),
    ]
    cp2 = [
        pltpu.async_copy(ei_hbm.at[0, pl.ds(eb + _H0 * 128, _H1 * 128)],
                         src_v.at[pl.ds(_H0 * 128, _H1 * 128)], sem_b),
        pltpu.async_copy(ei_hbm.at[1, pl.ds(eb + _H0 * 128, _H1 * 128)],
                         dst_v.at[pl.ds(_H0 * 128, _H1 * 128)], sem_b),
    ]
    for c in cp:
        c.wait()

    def make_body(t):
        ib = t * 128
        ob = t * 256
        for g in range(8):
            sv = src_v[pl.ds(ib + 16 * g, 16)]
            dv = dst_v[pl.ds(ib + 16 * g, 16)]
            ws = plsc.load_gather(q_v, [sv])
            wd = plsc.load_gather(q_v, [dv + _NP])
            a0, a1 = plsc.unpack(plsc.bitcast(ws, jnp.bfloat16),
                                 format=plsc.PackFormat.INTERLEAVED)
            c0, c1 = plsc.unpack(plsc.bitcast(wd, jnp.bfloat16),
                                 format=plsc.PackFormat.INTERLEAVED)
            out_v[pl.ds(ob + 16 * g, 16)] = a0 + c0
            out_v[pl.ds(ob + 128 + 16 * g, 16)] = a1 + c1

    plsc.parallel_loop(0, _H0, unroll=8)(make_body)
    out0 = pltpu.async_copy(out_v.at[pl.ds(0, _H0 * 256)],
                            out_hbm.at[pl.ds(tlo * 256, _H0 * 256)], sem_o)
    for c in cp2:
        c.wait()
    plsc.parallel_loop(_H0, _TPW, unroll=8)(make_body)
    out1 = pltpu.async_copy(out_v.at[pl.ds(_H0 * 256, _H1 * 256)],
                            out_hbm.at[pl.ds((tlo + _H0) * 256, _H1 * 256)],
                            sem_o)
    out0.wait()
    out1.wait()


def kernel(x, edge_index, W, b):
    q = pl.pallas_call(
        _proj_body,
        grid=(8,),
        in_specs=[
            pl.BlockSpec((2, 2 * D_FEAT), lambda i: (0, 0)),
            pl.BlockSpec((_NP // 8, D_FEAT), lambda i: (i, 0)),
            pl.BlockSpec(memory_space=pltpu.SMEM),
        ],
        out_specs=pl.BlockSpec((2, _NP // 8), lambda i: (0, i)),
        out_shape=jax.ShapeDtypeStruct((2, _NP), jnp.int32),
    )(W, x, b)
    out_flat = _edge_score(q, edge_index.astype(jnp.int32))
    return (out_flat.reshape(_NT, 2, 128).transpose(0, 2, 1)
            .reshape(N_EDGES, NUM_CLASSES))


# speedup vs baseline: 1.0993x; 1.0993x over previous
import functools

import jax
import jax.numpy as jnp
from jax import lax
from jax.experimental import pallas as pl
from jax.experimental.pallas import tpu as pltpu
from jax.experimental.pallas import tpu_sc as plsc

N_NODES = 10000
N_EDGES = 320000
D_FEAT = 128
NUM_CLASSES = 2

_NC, _NS, _L = 2, 16, 16
_NW = _NC * _NS
_NT = N_EDGES // 128
_TPW = 79
_H0 = 40
_H1 = _TPW - _H0
_NP = 10240


def _proj_body(w_ref, x_ref, b_ref, q_ref):
    dims = (((1,), (1,)), ((), ()))
    w4 = jnp.concatenate([w_ref[:, :D_FEAT], w_ref[:, D_FEAT:]], axis=0)
    d = lax.dot_general(w4, x_ref[...], dims,
                        preferred_element_type=jnp.float32)

    def _pack16(row):
        return lax.bitcast_convert_type(
            row.astype(jnp.bfloat16), jnp.uint16).astype(jnp.int32)

    s0 = _pack16(d[0:1] + b_ref[0])
    s1 = _pack16(d[1:2] + b_ref[1])
    t0 = _pack16(d[2:3])
    t1 = _pack16(d[3:4])
    q_ref[0:1, :] = s0 | (s1 << 16)
    q_ref[1:2, :] = t0 | (t1 << 16)


_mesh = plsc.VectorSubcoreMesh(core_axis_name="c", subcore_axis_name="s")


@functools.partial(
    pl.kernel,
    mesh=_mesh,
    compiler_params=pltpu.CompilerParams(needs_layout_passes=False),
    out_type=jax.ShapeDtypeStruct((2 * N_EDGES,), jnp.float32),
    scratch_types=[
        pltpu.VMEM((2 * _NP,), jnp.int32),
        pltpu.VMEM((_TPW * 128,), jnp.int32),
        pltpu.VMEM((_TPW * 128,), jnp.int32),
        pltpu.VMEM((_TPW * 256,), jnp.float32),
        pltpu.SemaphoreType.DMA,
        pltpu.SemaphoreType.DMA,
        pltpu.SemaphoreType.DMA,
    ],
)
def _edge_score(q_hbm, ei_hbm, out_hbm, q_v, src_v, dst_v, out_v,
                sem_a, sem_b, sem_o):
    wid = lax.axis_index("s") * _NC + lax.axis_index("c")
    tlo = wid * (_NT - _TPW) // (_NW - 1)
    eb = tlo * 128
    cp = [
        pltpu.async_copy(q_hbm.at[0, :], q_v.at[pl.ds(0, _NP)], sem_a),
        pltpu.async_copy(q_hbm.at[1, :], q_v.at[pl.ds(_NP, _NP)], sem_a),
        pltpu.async_copy(ei_hbm.at[0, pl.ds(eb, _H0 * 128)],
                         src_v.at[pl.ds(0, _H0 * 128)], sem_a),
        pltpu.async_copy(ei_hbm.at[1, pl.ds(eb, _H0 * 128)],
                         dst_v.at[pl.ds(0, _H0 * 128)], sem_a),
    ]
    cp2 = [
        pltpu.async_copy(ei_hbm.at[0, pl.ds(eb + _H0 * 128, _H1 * 128)],
                         src_v.at[pl.ds(_H0 * 128, _H1 * 128)], sem_b),
        pltpu.async_copy(ei_hbm.at[1, pl.ds(eb + _H0 * 128, _H1 * 128)],
                         dst_v.at[pl.ds(_H0 * 128, _H1 * 128)], sem_b),
    ]
    for c in cp:
        c.wait()

    def make_body(t):
        ib = t * 128
        ob = t * 256
        for g in range(8):
            sv = src_v[pl.ds(ib + 16 * g, 16)]
            dv = dst_v[pl.ds(ib + 16 * g, 16)]
            ws = plsc.load_gather(q_v, [sv])
            wd = plsc.load_gather(q_v, [dv + _NP])
            a0, a1 = plsc.unpack(plsc.bitcast(ws, jnp.bfloat16),
                                 format=plsc.PackFormat.INTERLEAVED)
            c0, c1 = plsc.unpack(plsc.bitcast(wd, jnp.bfloat16),
                                 format=plsc.PackFormat.INTERLEAVED)
            out_v[pl.ds(ob + 16 * g, 16)] = a0 + c0
            out_v[pl.ds(ob + 128 + 16 * g, 16)] = a1 + c1

    plsc.parallel_loop(0, _H0, unroll=4)(make_body)
    out0 = pltpu.async_copy(out_v.at[pl.ds(0, _H0 * 256)],
                            out_hbm.at[pl.ds(tlo * 256, _H0 * 256)], sem_o)
    for c in cp2:
        c.wait()
    plsc.parallel_loop(_H0, _TPW, unroll=4)(make_body)
    out1 = pltpu.async_copy(out_v.at[pl.ds(_H0 * 256, _H1 * 256)],
                            out_hbm.at[pl.ds((tlo + _H0) * 256, _H1 * 256)],
                            sem_o)
    out0.wait()
    out1.wait()


def kernel(x, edge_index, W, b):
    q = pl.pallas_call(
        _proj_body,
        grid=(4,),
        in_specs=[
            pl.BlockSpec((2, 2 * D_FEAT), lambda i: (0, 0)),
            pl.BlockSpec((_NP // 4, D_FEAT), lambda i: (i, 0)),
            pl.BlockSpec(memory_space=pltpu.SMEM),
        ],
        out_specs=pl.BlockSpec((2, _NP // 4), lambda i: (0, i)),
        out_shape=jax.ShapeDtypeStruct((2, _NP), jnp.int32),
    )(W, x, b)
    out_flat = _edge_score(q, edge_index.astype(jnp.int32))
    return (out_flat.reshape(_NT, 2, 128).transpose(0, 2, 1)
            .reshape(N_EDGES, NUM_CLASSES))
